# baseline (speedup 1.0000x reference)
import functools

import jax
import jax.numpy as jnp
from jax import lax
from jax.experimental import pallas as pl
from jax.experimental.pallas import tpu as pltpu
from jax.experimental.pallas import tpu_sc as plsc

N_NODES = 10000
N_EDGES = 320000
N_GRAPHS = 128
N_CLASSES = 10
HALF = N_NODES // 2

NCORES = 2
NSUB = 16
NW = NCORES * NSUB
CHUNK = 128
NCHUNK = 80
EPT = CHUNK * NCHUNK
N_EDGES_PAD = EPT * NW
ACC_ROWS = N_NODES + 16
RPT = N_NODES // NSUB

ROWS_B = 1000
GRID = HALF // ROWS_B


NBUF = 8


def _sc_segsum_body(p_hbm, zeros_hbm, edges_hbm, out_hbm,
                    srcv, dstv,
                    rows0, rows1, rows2, rows3, rows4, rows5, rows6, rows7,
                    acc,
                    g0, g1, g2, g3, g4, g5, g6, g7,
                    s0, s1, s2, s3, s4, s5, s6, s7):
    c = lax.axis_index("c")
    s = lax.axis_index("s")
    wid = c * NSUB + s
    rows = [rows0, rows1, rows2, rows3, rows4, rows5, rows6, rows7]
    gsem = [g0, g1, g2, g3, g4, g5, g6, g7]
    ssem = [s0, s1, s2, s3, s4, s5, s6, s7]

    pltpu.sync_copy(zeros_hbm.at[pl.ds(s * RPT, RPT)],
                    acc.at[pl.ds(s * RPT, RPT)])
    pltpu.sync_copy(edges_hbm.at[0, wid], srcv)
    pltpu.sync_copy(edges_hbm.at[1, wid], dstv)
    plsc.subcore_barrier()

    for b in range(NBUF):
        pltpu.async_copy(p_hbm.at[srcv.at[b]], rows[b], gsem[b])

    def step(k, carry):
        for b in range(NBUF):
            j = k * NBUF + b
            pltpu.make_async_copy(p_hbm.at[srcv.at[j]], rows[b],
                                  gsem[b]).wait()
            pltpu.async_copy(rows[b], acc.at[dstv.at[j]], ssem[b], add=True)
        for b in range(NBUF):
            j = k * NBUF + b
            jn = jnp.minimum(j + NBUF, NCHUNK - 1)
            pltpu.make_async_copy(rows[b], acc.at[dstv.at[j]],
                                  ssem[b]).wait()
            pltpu.async_copy(p_hbm.at[srcv.at[jn]], rows[b], gsem[b])
        return carry

    lax.fori_loop(0, NCHUNK // NBUF, step, 0)
    for b in range(NBUF):
        pltpu.make_async_copy(p_hbm.at[srcv.at[b]], rows[b], gsem[b]).wait()
    plsc.subcore_barrier()
    pltpu.sync_copy(acc.at[pl.ds(s * RPT, RPT)],
                    out_hbm.at[c, pl.ds(s * RPT, RPT)])


@functools.lru_cache(maxsize=None)
def _get_sc_segsum():
    return pl.kernel(
        _sc_segsum_body,
        out_type=jax.ShapeDtypeStruct((NCORES, N_NODES, 64), jnp.float32),
        mesh=plsc.VectorSubcoreMesh(core_axis_name="c", subcore_axis_name="s",
                                    num_cores=NCORES, num_subcores=NSUB),
        scratch_types=[
            pltpu.VMEM((NCHUNK, CHUNK), jnp.int32),
            pltpu.VMEM((NCHUNK, CHUNK), jnp.int32),
        ] + [pltpu.VMEM((CHUNK, 64), jnp.float32) for _ in range(NBUF)] + [
            pltpu.VMEM_SHARED((ACC_ROWS, 64), jnp.float32),
        ] + [pltpu.SemaphoreType.DMA for _ in range(2 * NBUF)],
        compiler_params=pltpu.CompilerParams(use_tc_tiling_on_sc=False),
    )


def _stage_a_body(xt_ref, xb_ref, w1_ref, p_ref):
    yt = jnp.dot(xt_ref[...], w1_ref[...], preferred_element_type=jnp.float32)
    yb = jnp.dot(xb_ref[...], w1_ref[...], preferred_element_type=jnp.float32)
    p_ref[...] = jnp.concatenate([yt, yb], axis=1)


def _stage_a(x, W1):
    return pl.pallas_call(
        _stage_a_body,
        grid=(GRID,),
        in_specs=[
            pl.BlockSpec((ROWS_B, 128), lambda i: (i, 0)),
            pl.BlockSpec((ROWS_B, 128), lambda i: (i + GRID, 0)),
            pl.BlockSpec((128, 64), lambda i: (0, 0)),
        ],
        out_specs=pl.BlockSpec((ROWS_B, 128), lambda i: (i, 0)),
        out_shape=jax.ShapeDtypeStruct((HALF, 128), jnp.float32),
    )(x, x, W1)


def _stage_b_body(parts_ref, p_ref, b1_ref, w2_ref, b2_ref, w3_ref, q_ref):
    S = parts_ref[0] + parts_ref[1] + p_ref[...]
    z1 = jnp.maximum(S + b1_ref[...], 0.0)
    h = jnp.maximum(
        jnp.dot(z1, w2_ref[...], preferred_element_type=jnp.float32)
        + b2_ref[...], 0.0)
    q_ref[...] = jnp.dot(h, w3_ref[...], preferred_element_type=jnp.float32)


def _stage_b(parts, p, b1p, W2bd, b2p, W3bd):
    return pl.pallas_call(
        _stage_b_body,
        grid=(GRID,),
        in_specs=[
            pl.BlockSpec((NCORES, ROWS_B, 128), lambda i: (0, i, 0)),
            pl.BlockSpec((ROWS_B, 128), lambda i: (i, 0)),
            pl.BlockSpec((1, 128), lambda i: (0, 0)),
            pl.BlockSpec((128, 128), lambda i: (0, 0)),
            pl.BlockSpec((1, 128), lambda i: (0, 0)),
            pl.BlockSpec((128, 128), lambda i: (0, 0)),
        ],
        out_specs=pl.BlockSpec((ROWS_B, 128), lambda i: (i, 0)),
        out_shape=jax.ShapeDtypeStruct((HALF, 128), jnp.float32),
    )(parts, p, b1p, W2bd, b2p, W3bd)


def _stage_c_body(parts_ref, q_ref, b3_ref, w4t_ref, w4b_ref, b4_ref,
                  bt_ref, bb_ref, wf_ref, bf_ref, o_ref, sums_ref, cnt_ref):
    i = pl.program_id(0)

    @pl.when(i == 0)
    def _():
        sums_ref[...] = jnp.zeros_like(sums_ref)
        cnt_ref[...] = jnp.zeros_like(cnt_ref)

    T = parts_ref[0] + parts_ref[1] + q_ref[...]
    z2 = jnp.maximum(T + b3_ref[...], 0.0)
    a_half = jnp.dot(z2, w4t_ref[...],
                     preferred_element_type=jnp.float32) + b4_ref[...]
    b_half = jnp.dot(z2, w4b_ref[...],
                     preferred_element_type=jnp.float32) + b4_ref[...]
    gids = lax.broadcasted_iota(jnp.int32, (ROWS_B, N_GRAPHS), 1)
    mt = (bt_ref[...] == gids).astype(jnp.float32)
    mb = (bb_ref[...] == gids).astype(jnp.float32)
    dn = (((0,), (0,)), ((), ()))
    sums_ref[...] += (
        lax.dot_general(mt, a_half, dn, preferred_element_type=jnp.float32)
        + lax.dot_general(mb, b_half, dn, preferred_element_type=jnp.float32))
    ones = jnp.ones((ROWS_B, 1), jnp.float32)
    cnt_ref[...] += (
        lax.dot_general(mt, ones, dn, preferred_element_type=jnp.float32)
        + lax.dot_general(mb, ones, dn, preferred_element_type=jnp.float32))

    @pl.when(i == GRID - 1)
    def _():
        pooled = sums_ref[...] / jnp.maximum(cnt_ref[...], 1.0)
        logits = jnp.dot(pooled, wf_ref[...],
                         preferred_element_type=jnp.float32) + bf_ref[...]
        m = jnp.max(logits, axis=1, keepdims=True)
        e = logits - m
        lse = jnp.log(jnp.sum(jnp.exp(e), axis=1, keepdims=True))
        o_ref[...] = e - lse


def _stage_c(parts, q, b3p, W4t, W4b, b4r, batchc, Wf, bfr):
    return pl.pallas_call(
        _stage_c_body,
        grid=(GRID,),
        in_specs=[
            pl.BlockSpec((NCORES, ROWS_B, 128), lambda i: (0, i, 0)),
            pl.BlockSpec((ROWS_B, 128), lambda i: (i, 0)),
            pl.BlockSpec((1, 128), lambda i: (0, 0)),
            pl.BlockSpec((128, 128), lambda i: (0, 0)),
            pl.BlockSpec((128, 128), lambda i: (0, 0)),
            pl.BlockSpec((1, 128), lambda i: (0, 0)),
            pl.BlockSpec((ROWS_B, 1), lambda i: (i, 0)),
            pl.BlockSpec((ROWS_B, 1), lambda i: (i + GRID, 0)),
            pl.BlockSpec((128, N_CLASSES), lambda i: (0, 0)),
            pl.BlockSpec((1, N_CLASSES), lambda i: (0, 0)),
        ],
        out_specs=pl.BlockSpec((N_GRAPHS, N_CLASSES), lambda i: (0, 0)),
        out_shape=jax.ShapeDtypeStruct((N_GRAPHS, N_CLASSES), jnp.float32),
        scratch_shapes=[
            pltpu.VMEM((N_GRAPHS, N_GRAPHS), jnp.float32),
            pltpu.VMEM((N_GRAPHS, 1), jnp.float32),
        ],
    )(parts, q, b3p, W4t, W4b, b4r, batchc, batchc, Wf, bfr)


def kernel(x, edge_index, batch, W1, b1, W2, b2, W3, b3, W4, b4, Wf, bf):
    edges_p = jnp.where(edge_index < HALF, 2 * edge_index,
                        2 * edge_index - (N_NODES - 1))
    npad = N_EDGES_PAD - N_EDGES
    cyc = jnp.arange(npad, dtype=jnp.int32) % 16
    pad = jnp.stack([cyc, N_NODES + cyc])
    edges_p = jnp.concatenate([edges_p, pad], axis=1)
    edges_p = edges_p.reshape(2, NW, NCHUNK, CHUNK)
    zeros = jnp.zeros((N_NODES, 64), jnp.float32)
    batchc = batch.reshape(N_NODES, 1)
    zw = jnp.zeros((64, 64), jnp.float32)
    z128 = jnp.zeros((64, 128), jnp.float32)
    W2bd = jnp.block([[W2, zw], [zw, W2]])
    W3bd = jnp.block([[W3, zw], [zw, W3]])
    W4t = jnp.concatenate([W4, z128], axis=0)
    W4b = jnp.concatenate([z128, W4], axis=0)
    b1p = jnp.concatenate([b1, b1]).reshape(1, 128)
    b2p = jnp.concatenate([b2, b2]).reshape(1, 128)
    b3p = jnp.concatenate([b3, b3]).reshape(1, 128)
    b4r = b4.reshape(1, 128)
    bfr = bf.reshape(1, N_CLASSES)

    sc_segsum = _get_sc_segsum()
    p = _stage_a(x, W1)
    parts1 = sc_segsum(p.reshape(N_NODES, 64), zeros, edges_p)
    q = _stage_b(parts1.reshape(NCORES, HALF, 128), p, b1p, W2bd, b2p, W3bd)
    parts2 = sc_segsum(q.reshape(N_NODES, 64), zeros, edges_p)
    return _stage_c(parts2.reshape(NCORES, HALF, 128), q, b3p, W4t, W4b, b4r,
                    batchc, Wf, bfr)


# speedup vs baseline: 1.0549x; 1.0549x over previous
import functools

import jax
import jax.numpy as jnp
from jax import lax
from jax.experimental import pallas as pl
from jax.experimental.pallas import tpu as pltpu
from jax.experimental.pallas import tpu_sc as plsc

N_NODES = 10000
N_EDGES = 320000
N_GRAPHS = 128
N_CLASSES = 10
HALF = N_NODES // 2

NCORES = 2
NSUB = 16
NW = NCORES * NSUB
CHUNK = 128
NCHUNK = 80
EPT = CHUNK * NCHUNK
N_EDGES_PAD = EPT * NW
ACC_ROWS = N_NODES + 16
RPT = N_NODES // NSUB

ROWS_B = 1000
GRID = HALF // ROWS_B


NBUF = 4


def _sc_segsum_body(p_hbm, zeros_hbm, edges_hbm, out_hbm,
                    srcv, dstv, rows0, rows1, rows2, rows3,
                    acc, sem0, sem1, sem2, sem3):
    c = lax.axis_index("c")
    s = lax.axis_index("s")
    wid = c * NSUB + s
    rows = [rows0, rows1, rows2, rows3]
    sems = [sem0, sem1, sem2, sem3]

    pltpu.sync_copy(zeros_hbm.at[pl.ds(s * RPT, RPT)],
                    acc.at[pl.ds(s * RPT, RPT)])
    pltpu.sync_copy(edges_hbm.at[0, wid], srcv)
    pltpu.sync_copy(edges_hbm.at[1, wid], dstv)
    plsc.subcore_barrier()

    for b in range(NBUF):
        pltpu.async_copy(p_hbm.at[srcv.at[b]], rows[b], sems[b])

    def step(k, carry):
        for b in range(NBUF):
            j = k * NBUF + b
            pltpu.make_async_copy(p_hbm.at[srcv.at[j]], rows[b],
                                  sems[b]).wait()
            pltpu.sync_copy(rows[b], acc.at[dstv.at[j]], add=True)
            jn = jnp.minimum(j + NBUF, NCHUNK - 1)
            pltpu.async_copy(p_hbm.at[srcv.at[jn]], rows[b], sems[b])
        return carry

    lax.fori_loop(0, NCHUNK // NBUF, step, 0)
    for b in range(NBUF):
        pltpu.make_async_copy(p_hbm.at[srcv.at[b]], rows[b], sems[b]).wait()
    plsc.subcore_barrier()
    pltpu.sync_copy(acc.at[pl.ds(s * RPT, RPT)],
                    out_hbm.at[c, pl.ds(s * RPT, RPT)])


@functools.lru_cache(maxsize=None)
def _get_sc_segsum():
    return pl.kernel(
        _sc_segsum_body,
        out_type=jax.ShapeDtypeStruct((NCORES, N_NODES, 64), jnp.float32),
        mesh=plsc.VectorSubcoreMesh(core_axis_name="c", subcore_axis_name="s",
                                    num_cores=NCORES, num_subcores=NSUB),
        scratch_types=[
            pltpu.VMEM((NCHUNK, CHUNK), jnp.int32),
            pltpu.VMEM((NCHUNK, CHUNK), jnp.int32),
        ] + [pltpu.VMEM((CHUNK, 64), jnp.float32) for _ in range(NBUF)] + [
            pltpu.VMEM_SHARED((ACC_ROWS, 64), jnp.float32),
        ] + [pltpu.SemaphoreType.DMA for _ in range(NBUF)],
        compiler_params=pltpu.CompilerParams(use_tc_tiling_on_sc=False),
    )


def _stage_a_body(xt_ref, xb_ref, w1_ref, p_ref):
    yt = jnp.dot(xt_ref[...], w1_ref[...], preferred_element_type=jnp.float32)
    yb = jnp.dot(xb_ref[...], w1_ref[...], preferred_element_type=jnp.float32)
    p_ref[...] = jnp.concatenate([yt, yb], axis=1)


def _stage_a(x, W1):
    return pl.pallas_call(
        _stage_a_body,
        grid=(GRID,),
        in_specs=[
            pl.BlockSpec((ROWS_B, 128), lambda i: (i, 0)),
            pl.BlockSpec((ROWS_B, 128), lambda i: (i + GRID, 0)),
            pl.BlockSpec((128, 64), lambda i: (0, 0)),
        ],
        out_specs=pl.BlockSpec((ROWS_B, 128), lambda i: (i, 0)),
        out_shape=jax.ShapeDtypeStruct((HALF, 128), jnp.float32),
    )(x, x, W1)


def _stage_b_body(parts_ref, p_ref, b1_ref, w2_ref, b2_ref, w3_ref, q_ref):
    S = parts_ref[0] + parts_ref[1] + p_ref[...]
    z1 = jnp.maximum(S + b1_ref[...], 0.0)
    h = jnp.maximum(
        jnp.dot(z1, w2_ref[...], preferred_element_type=jnp.float32)
        + b2_ref[...], 0.0)
    q_ref[...] = jnp.dot(h, w3_ref[...], preferred_element_type=jnp.float32)


def _stage_b(parts, p, b1p, W2bd, b2p, W3bd):
    return pl.pallas_call(
        _stage_b_body,
        grid=(GRID,),
        in_specs=[
            pl.BlockSpec((NCORES, ROWS_B, 128), lambda i: (0, i, 0)),
            pl.BlockSpec((ROWS_B, 128), lambda i: (i, 0)),
            pl.BlockSpec((1, 128), lambda i: (0, 0)),
            pl.BlockSpec((128, 128), lambda i: (0, 0)),
            pl.BlockSpec((1, 128), lambda i: (0, 0)),
            pl.BlockSpec((128, 128), lambda i: (0, 0)),
        ],
        out_specs=pl.BlockSpec((ROWS_B, 128), lambda i: (i, 0)),
        out_shape=jax.ShapeDtypeStruct((HALF, 128), jnp.float32),
    )(parts, p, b1p, W2bd, b2p, W3bd)


def _stage_c_body(parts_ref, q_ref, b3_ref, w4t_ref, w4b_ref, b4_ref,
                  bt_ref, bb_ref, wf_ref, bf_ref, o_ref, sums_ref, cnt_ref):
    i = pl.program_id(0)

    @pl.when(i == 0)
    def _():
        sums_ref[...] = jnp.zeros_like(sums_ref)
        cnt_ref[...] = jnp.zeros_like(cnt_ref)

    T = parts_ref[0] + parts_ref[1] + q_ref[...]
    z2 = jnp.maximum(T + b3_ref[...], 0.0)
    a_half = jnp.dot(z2, w4t_ref[...],
                     preferred_element_type=jnp.float32) + b4_ref[...]
    b_half = jnp.dot(z2, w4b_ref[...],
                     preferred_element_type=jnp.float32) + b4_ref[...]
    gids = lax.broadcasted_iota(jnp.int32, (ROWS_B, N_GRAPHS), 1)
    mt = (bt_ref[...] == gids).astype(jnp.float32)
    mb = (bb_ref[...] == gids).astype(jnp.float32)
    dn = (((0,), (0,)), ((), ()))
    sums_ref[...] += (
        lax.dot_general(mt, a_half, dn, preferred_element_type=jnp.float32)
        + lax.dot_general(mb, b_half, dn, preferred_element_type=jnp.float32))
    ones = jnp.ones((ROWS_B, 1), jnp.float32)
    cnt_ref[...] += (
        lax.dot_general(mt, ones, dn, preferred_element_type=jnp.float32)
        + lax.dot_general(mb, ones, dn, preferred_element_type=jnp.float32))

    @pl.when(i == GRID - 1)
    def _():
        pooled = sums_ref[...] / jnp.maximum(cnt_ref[...], 1.0)
        logits = jnp.dot(pooled, wf_ref[...],
                         preferred_element_type=jnp.float32) + bf_ref[...]
        m = jnp.max(logits, axis=1, keepdims=True)
        e = logits - m
        lse = jnp.log(jnp.sum(jnp.exp(e), axis=1, keepdims=True))
        o_ref[...] = e - lse


def _stage_c(parts, q, b3p, W4t, W4b, b4r, batchc, Wf, bfr):
    return pl.pallas_call(
        _stage_c_body,
        grid=(GRID,),
        in_specs=[
            pl.BlockSpec((NCORES, ROWS_B, 128), lambda i: (0, i, 0)),
            pl.BlockSpec((ROWS_B, 128), lambda i: (i, 0)),
            pl.BlockSpec((1, 128), lambda i: (0, 0)),
            pl.BlockSpec((128, 128), lambda i: (0, 0)),
            pl.BlockSpec((128, 128), lambda i: (0, 0)),
            pl.BlockSpec((1, 128), lambda i: (0, 0)),
            pl.BlockSpec((ROWS_B, 1), lambda i: (i, 0)),
            pl.BlockSpec((ROWS_B, 1), lambda i: (i + GRID, 0)),
            pl.BlockSpec((128, N_CLASSES), lambda i: (0, 0)),
            pl.BlockSpec((1, N_CLASSES), lambda i: (0, 0)),
        ],
        out_specs=pl.BlockSpec((N_GRAPHS, N_CLASSES), lambda i: (0, 0)),
        out_shape=jax.ShapeDtypeStruct((N_GRAPHS, N_CLASSES), jnp.float32),
        scratch_shapes=[
            pltpu.VMEM((N_GRAPHS, N_GRAPHS), jnp.float32),
            pltpu.VMEM((N_GRAPHS, 1), jnp.float32),
        ],
    )(parts, q, b3p, W4t, W4b, b4r, batchc, batchc, Wf, bfr)


def kernel(x, edge_index, batch, W1, b1, W2, b2, W3, b3, W4, b4, Wf, bf):
    edges_p = jnp.where(edge_index < HALF, 2 * edge_index,
                        2 * edge_index - (N_NODES - 1))
    npad = N_EDGES_PAD - N_EDGES
    cyc = jnp.arange(npad, dtype=jnp.int32) % 16
    pad = jnp.stack([cyc, N_NODES + cyc])
    edges_p = jnp.concatenate([edges_p, pad], axis=1)
    edges_p = edges_p.reshape(2, NW, NCHUNK, CHUNK)
    zeros = jnp.zeros((N_NODES, 64), jnp.float32)
    batchc = batch.reshape(N_NODES, 1)
    zw = jnp.zeros((64, 64), jnp.float32)
    z128 = jnp.zeros((64, 128), jnp.float32)
    W2bd = jnp.block([[W2, zw], [zw, W2]])
    W3bd = jnp.block([[W3, zw], [zw, W3]])
    W4t = jnp.concatenate([W4, z128], axis=0)
    W4b = jnp.concatenate([z128, W4], axis=0)
    b1p = jnp.concatenate([b1, b1]).reshape(1, 128)
    b2p = jnp.concatenate([b2, b2]).reshape(1, 128)
    b3p = jnp.concatenate([b3, b3]).reshape(1, 128)
    b4r = b4.reshape(1, 128)
    bfr = bf.reshape(1, N_CLASSES)

    sc_segsum = _get_sc_segsum()
    p = _stage_a(x, W1)
    parts1 = sc_segsum(p.reshape(N_NODES, 64), zeros, edges_p)
    q = _stage_b(parts1.reshape(NCORES, HALF, 128), p, b1p, W2bd, b2p, W3bd)
    parts2 = sc_segsum(q.reshape(N_NODES, 64), zeros, edges_p)
    return _stage_c(parts2.reshape(NCORES, HALF, 128), q, b3p, W4t, W4b, b4r,
                    batchc, Wf, bfr)


# speedup vs baseline: 1.0568x; 1.0018x over previous
import functools

import jax
import jax.numpy as jnp
from jax import lax
from jax.experimental import pallas as pl
from jax.experimental.pallas import tpu as pltpu
from jax.experimental.pallas import tpu_sc as plsc

N_NODES = 10000
N_EDGES = 320000
N_GRAPHS = 128
N_CLASSES = 10
HALF = N_NODES // 2

NCORES = 2
NSUB = 16
NW = NCORES * NSUB
CHUNK = 128
NCHUNK = 80
EPT = CHUNK * NCHUNK
N_EDGES_PAD = EPT * NW
ACC_ROWS = N_NODES + (EPT * NW - N_EDGES)
RPT = N_NODES // NSUB

ROWS_B = 1000
GRID = HALF // ROWS_B


NBUF = 4


def _sc_segsum_body(p_hbm, zeros_hbm, edges_hbm, out_hbm,
                    srcv, dstv, rows0, rows1, rows2, rows3,
                    acc, sem0, sem1, sem2, sem3):
    c = lax.axis_index("c")
    s = lax.axis_index("s")
    wid = c * NSUB + s
    rows = [rows0, rows1, rows2, rows3]
    sems = [sem0, sem1, sem2, sem3]

    pltpu.sync_copy(zeros_hbm.at[pl.ds(s * RPT, RPT)],
                    acc.at[pl.ds(s * RPT, RPT)])
    pltpu.sync_copy(edges_hbm.at[0, wid], srcv)
    pltpu.sync_copy(edges_hbm.at[1, wid], dstv)
    plsc.subcore_barrier()

    for b in range(NBUF):
        pltpu.async_copy(p_hbm.at[srcv.at[b]], rows[b], sems[b])

    def step(k, carry):
        for b in range(NBUF):
            j = k * NBUF + b
            pltpu.make_async_copy(p_hbm.at[srcv.at[j]], rows[b],
                                  sems[b]).wait()
            pltpu.sync_copy(rows[b], acc.at[dstv.at[j]], add=True)
            jn = jnp.minimum(j + NBUF, NCHUNK - 1)
            pltpu.async_copy(p_hbm.at[srcv.at[jn]], rows[b], sems[b])
        return carry

    lax.fori_loop(0, NCHUNK // NBUF, step, 0)
    for b in range(NBUF):
        pltpu.make_async_copy(p_hbm.at[srcv.at[b]], rows[b], sems[b]).wait()
    plsc.subcore_barrier()
    pltpu.sync_copy(acc.at[pl.ds(s * RPT, RPT)],
                    out_hbm.at[c, pl.ds(s * RPT, RPT)])


@functools.lru_cache(maxsize=None)
def _get_sc_segsum():
    return pl.kernel(
        _sc_segsum_body,
        out_type=jax.ShapeDtypeStruct((NCORES, N_NODES, 64), jnp.float32),
        mesh=plsc.VectorSubcoreMesh(core_axis_name="c", subcore_axis_name="s",
                                    num_cores=NCORES, num_subcores=NSUB),
        scratch_types=[
            pltpu.VMEM((NCHUNK, CHUNK), jnp.int32),
            pltpu.VMEM((NCHUNK, CHUNK), jnp.int32),
        ] + [pltpu.VMEM((CHUNK, 64), jnp.float32) for _ in range(NBUF)] + [
            pltpu.VMEM_SHARED((ACC_ROWS, 64), jnp.float32),
        ] + [pltpu.SemaphoreType.DMA for _ in range(NBUF)],
        compiler_params=pltpu.CompilerParams(use_tc_tiling_on_sc=False),
    )


def _stage_a_body(xt_ref, xb_ref, w1_ref, p_ref):
    yt = jnp.dot(xt_ref[...], w1_ref[...], preferred_element_type=jnp.float32)
    yb = jnp.dot(xb_ref[...], w1_ref[...], preferred_element_type=jnp.float32)
    p_ref[...] = jnp.concatenate([yt, yb], axis=1)


def _stage_a(x, W1):
    return pl.pallas_call(
        _stage_a_body,
        grid=(GRID,),
        in_specs=[
            pl.BlockSpec((ROWS_B, 128), lambda i: (i, 0)),
            pl.BlockSpec((ROWS_B, 128), lambda i: (i + GRID, 0)),
            pl.BlockSpec((128, 64), lambda i: (0, 0)),
        ],
        out_specs=pl.BlockSpec((ROWS_B, 128), lambda i: (i, 0)),
        out_shape=jax.ShapeDtypeStruct((HALF, 128), jnp.float32),
    )(x, x, W1)


def _stage_b_body(parts_ref, p_ref, b1_ref, w2_ref, b2_ref, w3_ref, q_ref):
    S = parts_ref[0] + parts_ref[1] + p_ref[...]
    z1 = jnp.maximum(S + b1_ref[...], 0.0)
    h = jnp.maximum(
        jnp.dot(z1, w2_ref[...], preferred_element_type=jnp.float32)
        + b2_ref[...], 0.0)
    q_ref[...] = jnp.dot(h, w3_ref[...], preferred_element_type=jnp.float32)


def _stage_b(parts, p, b1p, W2bd, b2p, W3bd):
    return pl.pallas_call(
        _stage_b_body,
        grid=(GRID,),
        in_specs=[
            pl.BlockSpec((NCORES, ROWS_B, 128), lambda i: (0, i, 0)),
            pl.BlockSpec((ROWS_B, 128), lambda i: (i, 0)),
            pl.BlockSpec((1, 128), lambda i: (0, 0)),
            pl.BlockSpec((128, 128), lambda i: (0, 0)),
            pl.BlockSpec((1, 128), lambda i: (0, 0)),
            pl.BlockSpec((128, 128), lambda i: (0, 0)),
        ],
        out_specs=pl.BlockSpec((ROWS_B, 128), lambda i: (i, 0)),
        out_shape=jax.ShapeDtypeStruct((HALF, 128), jnp.float32),
    )(parts, p, b1p, W2bd, b2p, W3bd)


def _stage_c_body(parts_ref, q_ref, b3_ref, w4t_ref, w4b_ref, b4_ref,
                  bt_ref, bb_ref, wf_ref, bf_ref, o_ref, sums_ref, cnt_ref):
    i = pl.program_id(0)

    @pl.when(i == 0)
    def _():
        sums_ref[...] = jnp.zeros_like(sums_ref)
        cnt_ref[...] = jnp.zeros_like(cnt_ref)

    T = parts_ref[0] + parts_ref[1] + q_ref[...]
    z2 = jnp.maximum(T + b3_ref[...], 0.0)
    a_half = jnp.dot(z2, w4t_ref[...],
                     preferred_element_type=jnp.float32) + b4_ref[...]
    b_half = jnp.dot(z2, w4b_ref[...],
                     preferred_element_type=jnp.float32) + b4_ref[...]
    gids = lax.broadcasted_iota(jnp.int32, (ROWS_B, N_GRAPHS), 1)
    mt = (bt_ref[...] == gids).astype(jnp.float32)
    mb = (bb_ref[...] == gids).astype(jnp.float32)
    dn = (((0,), (0,)), ((), ()))
    sums_ref[...] += (
        lax.dot_general(mt, a_half, dn, preferred_element_type=jnp.float32)
        + lax.dot_general(mb, b_half, dn, preferred_element_type=jnp.float32))
    ones = jnp.ones((ROWS_B, 1), jnp.float32)
    cnt_ref[...] += (
        lax.dot_general(mt, ones, dn, preferred_element_type=jnp.float32)
        + lax.dot_general(mb, ones, dn, preferred_element_type=jnp.float32))

    @pl.when(i == GRID - 1)
    def _():
        pooled = sums_ref[...] / jnp.maximum(cnt_ref[...], 1.0)
        logits = jnp.dot(pooled, wf_ref[...],
                         preferred_element_type=jnp.float32) + bf_ref[...]
        m = jnp.max(logits, axis=1, keepdims=True)
        e = logits - m
        lse = jnp.log(jnp.sum(jnp.exp(e), axis=1, keepdims=True))
        o_ref[...] = e - lse


def _stage_c(parts, q, b3p, W4t, W4b, b4r, batchc, Wf, bfr):
    return pl.pallas_call(
        _stage_c_body,
        grid=(GRID,),
        in_specs=[
            pl.BlockSpec((NCORES, ROWS_B, 128), lambda i: (0, i, 0)),
            pl.BlockSpec((ROWS_B, 128), lambda i: (i, 0)),
            pl.BlockSpec((1, 128), lambda i: (0, 0)),
            pl.BlockSpec((128, 128), lambda i: (0, 0)),
            pl.BlockSpec((128, 128), lambda i: (0, 0)),
            pl.BlockSpec((1, 128), lambda i: (0, 0)),
            pl.BlockSpec((ROWS_B, 1), lambda i: (i, 0)),
            pl.BlockSpec((ROWS_B, 1), lambda i: (i + GRID, 0)),
            pl.BlockSpec((128, N_CLASSES), lambda i: (0, 0)),
            pl.BlockSpec((1, N_CLASSES), lambda i: (0, 0)),
        ],
        out_specs=pl.BlockSpec((N_GRAPHS, N_CLASSES), lambda i: (0, 0)),
        out_shape=jax.ShapeDtypeStruct((N_GRAPHS, N_CLASSES), jnp.float32),
        scratch_shapes=[
            pltpu.VMEM((N_GRAPHS, N_GRAPHS), jnp.float32),
            pltpu.VMEM((N_GRAPHS, 1), jnp.float32),
        ],
    )(parts, q, b3p, W4t, W4b, b4r, batchc, batchc, Wf, bfr)


def kernel(x, edge_index, batch, W1, b1, W2, b2, W3, b3, W4, b4, Wf, bf):
    edges_p = jnp.where(edge_index < HALF, 2 * edge_index,
                        2 * edge_index - (N_NODES - 1))
    npad = N_EDGES_PAD - N_EDGES
    cyc = jnp.arange(npad, dtype=jnp.int32)
    pad = jnp.stack([cyc % 16, N_NODES + cyc])
    edges_p = jnp.concatenate([edges_p, pad], axis=1)
    edges_p = edges_p.reshape(2, NW, NCHUNK, CHUNK)
    zeros = jnp.zeros((N_NODES, 64), jnp.float32)
    batchc = batch.reshape(N_NODES, 1)
    zw = jnp.zeros((64, 64), jnp.float32)
    z128 = jnp.zeros((64, 128), jnp.float32)
    W2bd = jnp.block([[W2, zw], [zw, W2]])
    W3bd = jnp.block([[W3, zw], [zw, W3]])
    W4t = jnp.concatenate([W4, z128], axis=0)
    W4b = jnp.concatenate([z128, W4], axis=0)
    b1p = jnp.concatenate([b1, b1]).reshape(1, 128)
    b2p = jnp.concatenate([b2, b2]).reshape(1, 128)
    b3p = jnp.concatenate([b3, b3]).reshape(1, 128)
    b4r = b4.reshape(1, 128)
    bfr = bf.reshape(1, N_CLASSES)

    sc_segsum = _get_sc_segsum()
    p = _stage_a(x, W1)
    parts1 = sc_segsum(p.reshape(N_NODES, 64), zeros, edges_p)
    q = _stage_b(parts1.reshape(NCORES, HALF, 128), p, b1p, W2bd, b2p, W3bd)
    parts2 = sc_segsum(q.reshape(N_NODES, 64), zeros, edges_p)
    return _stage_c(parts2.reshape(NCORES, HALF, 128), q, b3p, W4t, W4b, b4r,
                    batchc, Wf, bfr)


# speedup vs baseline: 1.3073x; 1.2370x over previous
import functools

import jax
import jax.numpy as jnp
from jax import lax
from jax.experimental import pallas as pl
from jax.experimental.pallas import tpu as pltpu
from jax.experimental.pallas import tpu_sc as plsc

N_NODES = 10000
N_EDGES = 320000
N_GRAPHS = 128
N_CLASSES = 10
HALF = N_NODES // 2

NCORES = 2
NSUB = 16
NW = NCORES * NSUB
EPT = N_EDGES // NW
CHUNK = 125
NCHUNK = EPT // CHUNK
RPT = N_NODES // NSUB

ROWS_B = 1000
GRID = HALF // ROWS_B


NBUF = 4


def _sc_segsum_body(p_hbm, zeros_hbm, edges_hbm, out_hbm,
                    srcv, dstv, rows0, rows1, rows2, rows3,
                    acc, sem0, sem1, sem2, sem3):
    c = lax.axis_index("c")
    s = lax.axis_index("s")
    wid = c * NSUB + s
    rows = [rows0, rows1, rows2, rows3]
    sems = [sem0, sem1, sem2, sem3]

    pltpu.sync_copy(zeros_hbm.at[pl.ds(s * RPT, RPT)],
                    acc.at[pl.ds(s * RPT, RPT)])
    pltpu.sync_copy(edges_hbm.at[0, pl.ds(wid * NCHUNK, NCHUNK)], srcv)
    pltpu.sync_copy(edges_hbm.at[1, pl.ds(wid * NCHUNK, NCHUNK)], dstv)
    plsc.subcore_barrier()

    for b in range(NBUF):
        pltpu.async_copy(p_hbm.at[srcv.at[b]], rows[b], sems[b])

    def step(k, carry):
        for b in range(NBUF):
            j = k * NBUF + b
            pltpu.make_async_copy(p_hbm.at[srcv.at[j]], rows[b],
                                  sems[b]).wait()
            pltpu.sync_copy(rows[b], acc.at[dstv.at[j]], add=True)
            jn = jnp.minimum(j + NBUF, NCHUNK - 1)
            pltpu.async_copy(p_hbm.at[srcv.at[jn]], rows[b], sems[b])
        return carry

    lax.fori_loop(0, NCHUNK // NBUF, step, 0)
    for b in range(NBUF):
        pltpu.make_async_copy(p_hbm.at[srcv.at[b]], rows[b], sems[b]).wait()
    plsc.subcore_barrier()
    pltpu.sync_copy(acc.at[pl.ds(s * RPT, RPT)],
                    out_hbm.at[c, pl.ds(s * RPT, RPT)])


@functools.lru_cache(maxsize=None)
def _get_sc_segsum():
    return pl.kernel(
        _sc_segsum_body,
        out_type=jax.ShapeDtypeStruct((NCORES, N_NODES, 64), jnp.float32),
        mesh=plsc.VectorSubcoreMesh(core_axis_name="c", subcore_axis_name="s",
                                    num_cores=NCORES, num_subcores=NSUB),
        scratch_types=[
            pltpu.VMEM((NCHUNK, CHUNK), jnp.int32),
            pltpu.VMEM((NCHUNK, CHUNK), jnp.int32),
        ] + [pltpu.VMEM((CHUNK, 64), jnp.float32) for _ in range(NBUF)] + [
            pltpu.VMEM_SHARED((N_NODES, 64), jnp.float32),
        ] + [pltpu.SemaphoreType.DMA for _ in range(NBUF)],
        compiler_params=pltpu.CompilerParams(use_tc_tiling_on_sc=False),
    )


def _stage_a_body(xt_ref, xb_ref, w1_ref, p_ref):
    yt = jnp.dot(xt_ref[...], w1_ref[...], preferred_element_type=jnp.float32)
    yb = jnp.dot(xb_ref[...], w1_ref[...], preferred_element_type=jnp.float32)
    p_ref[...] = jnp.concatenate([yt, yb], axis=1)


def _stage_a(x, W1):
    return pl.pallas_call(
        _stage_a_body,
        grid=(GRID,),
        in_specs=[
            pl.BlockSpec((ROWS_B, 128), lambda i: (i, 0)),
            pl.BlockSpec((ROWS_B, 128), lambda i: (i + GRID, 0)),
            pl.BlockSpec((128, 64), lambda i: (0, 0)),
        ],
        out_specs=pl.BlockSpec((ROWS_B, 128), lambda i: (i, 0)),
        out_shape=jax.ShapeDtypeStruct((HALF, 128), jnp.float32),
    )(x, x, W1)


def _stage_b_body(parts_ref, p_ref, b1_ref, w2_ref, b2_ref, w3_ref, q_ref):
    S = parts_ref[0] + parts_ref[1] + p_ref[...]
    z1 = jnp.maximum(S + b1_ref[...], 0.0)
    h = jnp.maximum(
        jnp.dot(z1, w2_ref[...], preferred_element_type=jnp.float32)
        + b2_ref[...], 0.0)
    q_ref[...] = jnp.dot(h, w3_ref[...], preferred_element_type=jnp.float32)


def _stage_b(parts, p, b1p, W2bd, b2p, W3bd):
    return pl.pallas_call(
        _stage_b_body,
        grid=(GRID,),
        in_specs=[
            pl.BlockSpec((NCORES, ROWS_B, 128), lambda i: (0, i, 0)),
            pl.BlockSpec((ROWS_B, 128), lambda i: (i, 0)),
            pl.BlockSpec((1, 128), lambda i: (0, 0)),
            pl.BlockSpec((128, 128), lambda i: (0, 0)),
            pl.BlockSpec((1, 128), lambda i: (0, 0)),
            pl.BlockSpec((128, 128), lambda i: (0, 0)),
        ],
        out_specs=pl.BlockSpec((ROWS_B, 128), lambda i: (i, 0)),
        out_shape=jax.ShapeDtypeStruct((HALF, 128), jnp.float32),
    )(parts, p, b1p, W2bd, b2p, W3bd)


def _stage_c_body(parts_ref, q_ref, b3_ref, w4t_ref, w4b_ref, b4_ref,
                  bt_ref, bb_ref, wf_ref, bf_ref, o_ref, sums_ref, cnt_ref):
    i = pl.program_id(0)

    @pl.when(i == 0)
    def _():
        sums_ref[...] = jnp.zeros_like(sums_ref)
        cnt_ref[...] = jnp.zeros_like(cnt_ref)

    T = parts_ref[0] + parts_ref[1] + q_ref[...]
    z2 = jnp.maximum(T + b3_ref[...], 0.0)
    a_half = jnp.dot(z2, w4t_ref[...],
                     preferred_element_type=jnp.float32) + b4_ref[...]
    b_half = jnp.dot(z2, w4b_ref[...],
                     preferred_element_type=jnp.float32) + b4_ref[...]
    gids = lax.broadcasted_iota(jnp.int32, (ROWS_B, N_GRAPHS), 1)
    mt = (bt_ref[...] == gids).astype(jnp.float32)
    mb = (bb_ref[...] == gids).astype(jnp.float32)
    dn = (((0,), (0,)), ((), ()))
    sums_ref[...] += (
        lax.dot_general(mt, a_half, dn, preferred_element_type=jnp.float32)
        + lax.dot_general(mb, b_half, dn, preferred_element_type=jnp.float32))
    ones = jnp.ones((ROWS_B, 1), jnp.float32)
    cnt_ref[...] += (
        lax.dot_general(mt, ones, dn, preferred_element_type=jnp.float32)
        + lax.dot_general(mb, ones, dn, preferred_element_type=jnp.float32))

    @pl.when(i == GRID - 1)
    def _():
        pooled = sums_ref[...] / jnp.maximum(cnt_ref[...], 1.0)
        logits = jnp.dot(pooled, wf_ref[...],
                         preferred_element_type=jnp.float32) + bf_ref[...]
        m = jnp.max(logits, axis=1, keepdims=True)
        e = logits - m
        lse = jnp.log(jnp.sum(jnp.exp(e), axis=1, keepdims=True))
        o_ref[...] = e - lse


def _stage_c(parts, q, b3p, W4t, W4b, b4r, batchc, Wf, bfr):
    return pl.pallas_call(
        _stage_c_body,
        grid=(GRID,),
        in_specs=[
            pl.BlockSpec((NCORES, ROWS_B, 128), lambda i: (0, i, 0)),
            pl.BlockSpec((ROWS_B, 128), lambda i: (i, 0)),
            pl.BlockSpec((1, 128), lambda i: (0, 0)),
            pl.BlockSpec((128, 128), lambda i: (0, 0)),
            pl.BlockSpec((128, 128), lambda i: (0, 0)),
            pl.BlockSpec((1, 128), lambda i: (0, 0)),
            pl.BlockSpec((ROWS_B, 1), lambda i: (i, 0)),
            pl.BlockSpec((ROWS_B, 1), lambda i: (i + GRID, 0)),
            pl.BlockSpec((128, N_CLASSES), lambda i: (0, 0)),
            pl.BlockSpec((1, N_CLASSES), lambda i: (0, 0)),
        ],
        out_specs=pl.BlockSpec((N_GRAPHS, N_CLASSES), lambda i: (0, 0)),
        out_shape=jax.ShapeDtypeStruct((N_GRAPHS, N_CLASSES), jnp.float32),
        scratch_shapes=[
            pltpu.VMEM((N_GRAPHS, N_GRAPHS), jnp.float32),
            pltpu.VMEM((N_GRAPHS, 1), jnp.float32),
        ],
    )(parts, q, b3p, W4t, W4b, b4r, batchc, batchc, Wf, bfr)


def kernel(x, edge_index, batch, W1, b1, W2, b2, W3, b3, W4, b4, Wf, bf):
    edges_p = jnp.where(edge_index < HALF, 2 * edge_index,
                        2 * edge_index - (N_NODES - 1))
    edges_p = edges_p.reshape(2, NW * NCHUNK, CHUNK)
    zeros = jnp.zeros((N_NODES, 64), jnp.float32)
    batchc = batch.reshape(N_NODES, 1)
    zw = jnp.zeros((64, 64), jnp.float32)
    z128 = jnp.zeros((64, 128), jnp.float32)
    W2bd = jnp.block([[W2, zw], [zw, W2]])
    W3bd = jnp.block([[W3, zw], [zw, W3]])
    W4t = jnp.concatenate([W4, z128], axis=0)
    W4b = jnp.concatenate([z128, W4], axis=0)
    b1p = jnp.concatenate([b1, b1]).reshape(1, 128)
    b2p = jnp.concatenate([b2, b2]).reshape(1, 128)
    b3p = jnp.concatenate([b3, b3]).reshape(1, 128)
    b4r = b4.reshape(1, 128)
    bfr = bf.reshape(1, N_CLASSES)

    sc_segsum = _get_sc_segsum()
    p = _stage_a(x, W1)
    parts1 = sc_segsum(p.reshape(N_NODES, 64), zeros, edges_p)
    q = _stage_b(parts1.reshape(NCORES, HALF, 128), p, b1p, W2bd, b2p, W3bd)
    parts2 = sc_segsum(q.reshape(N_NODES, 64), zeros, edges_p)
    return _stage_c(parts2.reshape(NCORES, HALF, 128), q, b3p, W4t, W4b, b4r,
                    batchc, Wf, bfr)


# speedup vs baseline: 1.3457x; 1.0294x over previous
import functools

import jax
import jax.numpy as jnp
from jax import lax
from jax.experimental import pallas as pl
from jax.experimental.pallas import tpu as pltpu
from jax.experimental.pallas import tpu_sc as plsc

N_NODES = 10000
N_EDGES = 320000
N_GRAPHS = 128
N_CLASSES = 10
HALF = N_NODES // 2

NCORES = 2
NSUB = 16
NW = NCORES * NSUB
EPT = N_EDGES // NW
CHUNK = 125
NCHUNK = EPT // CHUNK
RPT = N_NODES // NSUB

ROWS_B = 1000
GRID = HALF // ROWS_B


NBUF = 4


def _sc_segsum_body(p_hbm, edges_hbm, out_hbm,
                    srcv, dstv, rows0, rows1, rows2, rows3,
                    acc, sem0, sem1, sem2, sem3):
    c = lax.axis_index("c")
    s = lax.axis_index("s")
    wid = c * NSUB + s
    rows = [rows0, rows1, rows2, rows3]
    sems = [sem0, sem1, sem2, sem3]

    def zstep(i, carry):
        for part in range(4):
            rows0[i, pl.ds(part * 16, 16)] = jnp.zeros((16,), jnp.float32)
        return carry

    lax.fori_loop(0, CHUNK, zstep, 0)
    for m in range(RPT // CHUNK):
        pltpu.sync_copy(rows0, acc.at[pl.ds(s * RPT + m * CHUNK, CHUNK)])
    pltpu.sync_copy(edges_hbm.at[0, pl.ds(wid * NCHUNK, NCHUNK)], srcv)
    pltpu.sync_copy(edges_hbm.at[1, pl.ds(wid * NCHUNK, NCHUNK)], dstv)
    plsc.subcore_barrier()

    for b in range(NBUF):
        pltpu.async_copy(p_hbm.at[srcv.at[b]], rows[b], sems[b])

    def step(k, carry):
        for b in range(NBUF):
            j = k * NBUF + b
            pltpu.make_async_copy(p_hbm.at[srcv.at[j]], rows[b],
                                  sems[b]).wait()
            pltpu.sync_copy(rows[b], acc.at[dstv.at[j]], add=True)
            jn = jnp.minimum(j + NBUF, NCHUNK - 1)
            pltpu.async_copy(p_hbm.at[srcv.at[jn]], rows[b], sems[b])
        return carry

    lax.fori_loop(0, NCHUNK // NBUF, step, 0)
    for b in range(NBUF):
        pltpu.make_async_copy(p_hbm.at[srcv.at[b]], rows[b], sems[b]).wait()
    plsc.subcore_barrier()
    pltpu.sync_copy(acc.at[pl.ds(s * RPT, RPT)],
                    out_hbm.at[c, pl.ds(s * RPT, RPT)])


@functools.lru_cache(maxsize=None)
def _get_sc_segsum():
    return pl.kernel(
        _sc_segsum_body,
        out_type=jax.ShapeDtypeStruct((NCORES, N_NODES, 64), jnp.float32),
        mesh=plsc.VectorSubcoreMesh(core_axis_name="c", subcore_axis_name="s",
                                    num_cores=NCORES, num_subcores=NSUB),
        scratch_types=[
            pltpu.VMEM((NCHUNK, CHUNK), jnp.int32),
            pltpu.VMEM((NCHUNK, CHUNK), jnp.int32),
        ] + [pltpu.VMEM((CHUNK, 64), jnp.float32) for _ in range(NBUF)] + [
            pltpu.VMEM_SHARED((N_NODES, 64), jnp.float32),
        ] + [pltpu.SemaphoreType.DMA for _ in range(NBUF)],
        compiler_params=pltpu.CompilerParams(use_tc_tiling_on_sc=False),
    )


def _stage_a_body(xt_ref, xb_ref, w1_ref, p_ref):
    yt = jnp.dot(xt_ref[...], w1_ref[...], preferred_element_type=jnp.float32)
    yb = jnp.dot(xb_ref[...], w1_ref[...], preferred_element_type=jnp.float32)
    p_ref[...] = jnp.concatenate([yt, yb], axis=1)


def _stage_a(x, W1):
    return pl.pallas_call(
        _stage_a_body,
        grid=(GRID,),
        in_specs=[
            pl.BlockSpec((ROWS_B, 128), lambda i: (i, 0)),
            pl.BlockSpec((ROWS_B, 128), lambda i: (i + GRID, 0)),
            pl.BlockSpec((128, 64), lambda i: (0, 0)),
        ],
        out_specs=pl.BlockSpec((ROWS_B, 128), lambda i: (i, 0)),
        out_shape=jax.ShapeDtypeStruct((HALF, 128), jnp.float32),
    )(x, x, W1)


def _stage_b_body(parts_ref, p_ref, b1_ref, w2_ref, b2_ref, w3_ref, q_ref):
    S = parts_ref[0] + parts_ref[1] + p_ref[...]
    z1 = jnp.maximum(S + b1_ref[...], 0.0)
    h = jnp.maximum(
        jnp.dot(z1, w2_ref[...], preferred_element_type=jnp.float32)
        + b2_ref[...], 0.0)
    q_ref[...] = jnp.dot(h, w3_ref[...], preferred_element_type=jnp.float32)


def _stage_b(parts, p, b1p, W2bd, b2p, W3bd):
    return pl.pallas_call(
        _stage_b_body,
        grid=(GRID,),
        in_specs=[
            pl.BlockSpec((NCORES, ROWS_B, 128), lambda i: (0, i, 0)),
            pl.BlockSpec((ROWS_B, 128), lambda i: (i, 0)),
            pl.BlockSpec((1, 128), lambda i: (0, 0)),
            pl.BlockSpec((128, 128), lambda i: (0, 0)),
            pl.BlockSpec((1, 128), lambda i: (0, 0)),
            pl.BlockSpec((128, 128), lambda i: (0, 0)),
        ],
        out_specs=pl.BlockSpec((ROWS_B, 128), lambda i: (i, 0)),
        out_shape=jax.ShapeDtypeStruct((HALF, 128), jnp.float32),
    )(parts, p, b1p, W2bd, b2p, W3bd)


def _stage_c_body(parts_ref, q_ref, b3_ref, w4t_ref, w4b_ref, b4_ref,
                  bt_ref, bb_ref, wf_ref, bf_ref, o_ref, sums_ref, cnt_ref):
    i = pl.program_id(0)

    @pl.when(i == 0)
    def _():
        sums_ref[...] = jnp.zeros_like(sums_ref)
        cnt_ref[...] = jnp.zeros_like(cnt_ref)

    T = parts_ref[0] + parts_ref[1] + q_ref[...]
    z2 = jnp.maximum(T + b3_ref[...], 0.0)
    a_half = jnp.dot(z2, w4t_ref[...],
                     preferred_element_type=jnp.float32) + b4_ref[...]
    b_half = jnp.dot(z2, w4b_ref[...],
                     preferred_element_type=jnp.float32) + b4_ref[...]
    gids = lax.broadcasted_iota(jnp.int32, (ROWS_B, N_GRAPHS), 1)
    mt = (bt_ref[...] == gids).astype(jnp.float32)
    mb = (bb_ref[...] == gids).astype(jnp.float32)
    dn = (((0,), (0,)), ((), ()))
    sums_ref[...] += (
        lax.dot_general(mt, a_half, dn, preferred_element_type=jnp.float32)
        + lax.dot_general(mb, b_half, dn, preferred_element_type=jnp.float32))
    ones = jnp.ones((ROWS_B, 1), jnp.float32)
    cnt_ref[...] += (
        lax.dot_general(mt, ones, dn, preferred_element_type=jnp.float32)
        + lax.dot_general(mb, ones, dn, preferred_element_type=jnp.float32))

    @pl.when(i == GRID - 1)
    def _():
        pooled = sums_ref[...] / jnp.maximum(cnt_ref[...], 1.0)
        logits = jnp.dot(pooled, wf_ref[...],
                         preferred_element_type=jnp.float32) + bf_ref[...]
        m = jnp.max(logits, axis=1, keepdims=True)
        e = logits - m
        lse = jnp.log(jnp.sum(jnp.exp(e), axis=1, keepdims=True))
        o_ref[...] = e - lse


def _stage_c(parts, q, b3p, W4t, W4b, b4r, batchc, Wf, bfr):
    return pl.pallas_call(
        _stage_c_body,
        grid=(GRID,),
        in_specs=[
            pl.BlockSpec((NCORES, ROWS_B, 128), lambda i: (0, i, 0)),
            pl.BlockSpec((ROWS_B, 128), lambda i: (i, 0)),
            pl.BlockSpec((1, 128), lambda i: (0, 0)),
            pl.BlockSpec((128, 128), lambda i: (0, 0)),
            pl.BlockSpec((128, 128), lambda i: (0, 0)),
            pl.BlockSpec((1, 128), lambda i: (0, 0)),
            pl.BlockSpec((ROWS_B, 1), lambda i: (i, 0)),
            pl.BlockSpec((ROWS_B, 1), lambda i: (i + GRID, 0)),
            pl.BlockSpec((128, N_CLASSES), lambda i: (0, 0)),
            pl.BlockSpec((1, N_CLASSES), lambda i: (0, 0)),
        ],
        out_specs=pl.BlockSpec((N_GRAPHS, N_CLASSES), lambda i: (0, 0)),
        out_shape=jax.ShapeDtypeStruct((N_GRAPHS, N_CLASSES), jnp.float32),
        scratch_shapes=[
            pltpu.VMEM((N_GRAPHS, N_GRAPHS), jnp.float32),
            pltpu.VMEM((N_GRAPHS, 1), jnp.float32),
        ],
    )(parts, q, b3p, W4t, W4b, b4r, batchc, batchc, Wf, bfr)


def kernel(x, edge_index, batch, W1, b1, W2, b2, W3, b3, W4, b4, Wf, bf):
    edges_p = jnp.where(edge_index < HALF, 2 * edge_index,
                        2 * edge_index - (N_NODES - 1))
    edges_p = edges_p.reshape(2, NW * NCHUNK, CHUNK)
    batchc = batch.reshape(N_NODES, 1)
    zw = jnp.zeros((64, 64), jnp.float32)
    z128 = jnp.zeros((64, 128), jnp.float32)
    W2bd = jnp.block([[W2, zw], [zw, W2]])
    W3bd = jnp.block([[W3, zw], [zw, W3]])
    W4t = jnp.concatenate([W4, z128], axis=0)
    W4b = jnp.concatenate([z128, W4], axis=0)
    b1p = jnp.concatenate([b1, b1]).reshape(1, 128)
    b2p = jnp.concatenate([b2, b2]).reshape(1, 128)
    b3p = jnp.concatenate([b3, b3]).reshape(1, 128)
    b4r = b4.reshape(1, 128)
    bfr = bf.reshape(1, N_CLASSES)

    sc_segsum = _get_sc_segsum()
    p = _stage_a(x, W1)
    parts1 = sc_segsum(p.reshape(N_NODES, 64), edges_p)
    q = _stage_b(parts1.reshape(NCORES, HALF, 128), p, b1p, W2bd, b2p, W3bd)
    parts2 = sc_segsum(q.reshape(N_NODES, 64), edges_p)
    return _stage_c(parts2.reshape(NCORES, HALF, 128), q, b3p, W4t, W4b, b4r,
                    batchc, Wf, bfr)
